# baseline (speedup 1.0000x reference)
import functools

import jax
import jax.numpy as jnp
from jax import lax
from jax.experimental import pallas as pl
from jax.experimental.pallas import tpu as pltpu
from jax.experimental.pallas import tpu_sc as plsc

_LANE = 16
_CV = 5120


@functools.lru_cache(maxsize=None)
def _make_gather(n_fields, f_base, embed, vocab, batch):
    info = plsc.get_sparse_core_info()
    nc, ns = info.num_cores, info.num_subcores
    nw = nc * ns
    egroups = embed // 8
    n_units = n_fields * egroups
    units_per_w = -(-n_units // nw)
    v_aligned = (vocab // 128) * 128
    n_full = v_aligned // _CV
    tail_v = v_aligned - n_full * _CV
    n_chunks = n_full + (1 if tail_v else 0)
    pos_bits = batch.bit_length() - 1
    assert (1 << pos_bits) == batch and vocab * batch < 2**31

    mesh = plsc.VectorSubcoreMesh(core_axis_name="c", subcore_axis_name="s")

    @functools.partial(
        pl.kernel,
        mesh=mesh,
        compiler_params=pltpu.CompilerParams(
            use_tc_tiling_on_sc=True, needs_layout_passes=False),
        out_type=jax.ShapeDtypeStruct((n_fields * embed, batch), jnp.float32),
        scratch_types=[
            pltpu.VMEM((batch,), jnp.int32),
            pltpu.VMEM((32,), jnp.int32),
            pltpu.VMEM((_CV // 128, 8, 128), jnp.float32),
            pltpu.VMEM((_CV // 128, 8, 128), jnp.float32),
            pltpu.VMEM((1, 8, 128), jnp.float32),
            pltpu.VMEM((8, batch), jnp.float32),
            pltpu.SemaphoreType.DMA,
            pltpu.SemaphoreType.DMA,
        ],
    )
    def gather_k(tbl_hbm, reg_hbm, sv_hbm, bnd_hbm, out_hbm,
                 sv_v, bnd_v, buf0, buf1, reg_v, slab_v, sem0, sem1):
        wid = lax.axis_index("s") * nc + lax.axis_index("c")
        bufs = (buf0, buf1)
        sems = (sem0, sem1)
        iota16 = lax.iota(jnp.int32, _LANE)

        def tile_copy(f, g8, c, t):
            v0 = pl.multiple_of(c * _CV + t * 128, 128)
            return pltpu.make_async_copy(
                tbl_hbm.at[f, pl.ds(g8, 8), pl.ds(v0, 128)],
                bufs[c % 2].at[t], sems[c % 2])

        def fire(f, g8, c):
            nt = (_CV if c < n_full else tail_v) // 128
            lax.fori_loop(0, nt, lambda t, a: (tile_copy(f, g8, c, t).start(), a)[1], 0)

        def wait(f, g8, c):
            nt = (_CV if c < n_full else tail_v) // 128
            lax.fori_loop(0, nt, lambda t, a: (tile_copy(f, g8, c, t).wait(), a)[1], 0)

        def scalar_at(vec_ref, j):
            grp = vec_ref[pl.ds((j // _LANE) * _LANE, _LANE)]
            return jnp.sum(jnp.where(iota16 == (j % _LANE), grp, 0))

        def unit_body(j, carry):
            u = wid + j * nw

            @pl.when(u < n_units)
            def _():
                fl = u // egroups
                f = fl + f_base
                g8 = pl.multiple_of((u % egroups) * 8, 8)
                pltpu.sync_copy(sv_hbm.at[fl], sv_v)
                pltpu.sync_copy(bnd_hbm.at[fl], bnd_v)
                pltpu.sync_copy(reg_hbm.at[fl, pl.ds(g8, 8), :], reg_v.at[0])

                def process(buf, v0, v1, lo, hi):
                    def group_body(k, c2):
                        off = pl.multiple_of(k * _LANE, 8)
                        pk16 = sv_v[pl.ds(off, _LANE)]
                        sv16 = lax.shift_right_logical(pk16, pos_bits)
                        pos16 = jnp.bitwise_and(pk16, batch - 1)
                        m = jnp.logical_and(sv16 >= v0, sv16 < v1)
                        vloc = jnp.where(m, sv16 - v0, 0)
                        t16 = lax.shift_right_logical(vloc, 7)
                        l16 = jnp.bitwise_and(vloc, 127)
                        for s in range(8):
                            s16 = jnp.full((_LANE,), s, jnp.int32)
                            vals = plsc.load_gather(buf, [t16, s16, l16])
                            plsc.store_scatter(slab_v, [s16, pos16], vals, mask=m)
                        return c2

                    lax.fori_loop(lo >> 4, (hi + _LANE - 1) >> 4, group_body, 0)

                fire(f, g8, 0)
                for c in range(n_chunks):
                    if c + 1 < n_chunks:
                        fire(f, g8, c + 1)
                    wait(f, g8, c)
                    v0 = c * _CV
                    v1 = v0 + (_CV if c < n_full else tail_v)
                    process(bufs[c % 2], v0, v1,
                            scalar_at(bnd_v, c), scalar_at(bnd_v, c + 1))
                process(reg_v, v_aligned, vocab,
                        scalar_at(bnd_v, n_chunks), scalar_at(bnd_v, n_chunks + 1))
                row0 = fl * embed + g8
                pltpu.sync_copy(slab_v, out_hbm.at[pl.ds(row0, 8), :])

            return carry

        lax.fori_loop(0, units_per_w, unit_body, 0)

    return gather_k


def _mlp_t_body(*refs):
    rt0_ref, rt1_ref = refs[0], refs[1]
    out_ref = refs[-1]
    w = refs[2:-1]
    rt = jnp.concatenate([rt0_ref[...], rt1_ref[...]], axis=0)
    cdims = (((0,), (0,)), ((), ()))

    def dot(a, b):
        return lax.dot_general(a, b, cdims, preferred_element_type=jnp.float32)

    n_units = (len(w) - 2) // 4
    for u in range(n_units):
        w1, b1, w2, b2 = w[4 * u : 4 * u + 4]
        h = jnp.maximum(dot(w1[...], rt) + b1[...], 0.0)
        h = dot(w2[...], h)
        rt = jnp.maximum(rt + h + b2[...], 0.0)
    wd, bd = w[-2], w[-1]
    logit = dot(wd[...], rt)
    out_ref[...] = jax.nn.sigmoid(logit + bd[...])


def _mlp_t(rt0, rt1, flat_w, block_c=512):
    batch = rt0.shape[1]
    grid = (batch // block_c,)
    full = lambda a: pl.BlockSpec(a.shape, lambda i: (0,) * a.ndim)
    in_specs = [pl.BlockSpec((rt0.shape[0], block_c), lambda i: (0, i)),
                pl.BlockSpec((rt1.shape[0], block_c), lambda i: (0, i))]
    in_specs += [full(a) for a in flat_w]
    return pl.pallas_call(
        _mlp_t_body,
        grid=grid,
        in_specs=in_specs,
        out_specs=pl.BlockSpec((1, block_c), lambda i: (0, i)),
        out_shape=jax.ShapeDtypeStruct((1, batch), jnp.float32),
    )(rt0, rt1, *flat_w)


def kernel(sparse_inputs, params):
    tables = params["tables"]
    n_fields, vocab, embed = tables.shape
    batch = sparse_inputs.shape[0]
    tbl_t = jnp.transpose(tables, (0, 2, 1))

    idx_t = sparse_inputs.astype(jnp.int32).T
    iota_b = jnp.broadcast_to(
        jnp.arange(batch, dtype=jnp.int32)[None, :], idx_t.shape)
    packed = idx_t * batch + iota_b
    v_aligned = (vocab // 128) * 128
    grid = jnp.array(
        list(range(0, v_aligned + 1, _CV))[: v_aligned // _CV + 1]
        + [v_aligned, vocab], dtype=jnp.int32)
    bnd = jnp.sum(idx_t[:, :, None] < grid[None, None, :], axis=1,
                  dtype=jnp.int32)
    bnd = jnp.pad(bnd, ((0, 0), (0, 32 - bnd.shape[1])), mode="edge")
    reg = jnp.pad(tbl_t[:, :, v_aligned:], ((0, 0), (0, 0),
                                            (0, 128 - (vocab - v_aligned))))

    fh = max(1, n_fields // 6)
    flat_w = []
    for (w1, b1, w2, b2) in params["res"]:
        flat_w += [w1, b1[:, None], w2, b2[:, None]]
    flat_w += [params["Wd"], params["bd"][:, None]]

    halves = []
    for f0, f1 in ((0, fh), (fh, n_fields)):
        sv_h = lax.sort(packed[f0:f1], dimension=1)
        halves.append(_make_gather(f1 - f0, f0, embed, vocab, batch)(
            tbl_t, reg[f0:f1], sv_h, bnd[f0:f1]))
    out_t = _mlp_t(halves[0], halves[1], tuple(flat_w))
    return out_t.reshape(batch, 1)


# speedup vs baseline: 1.0977x; 1.0977x over previous
import functools

import jax
import jax.numpy as jnp
from jax import lax
from jax.experimental import pallas as pl
from jax.experimental.pallas import tpu as pltpu
from jax.experimental.pallas import tpu_sc as plsc

_LANE = 16
_CV = 5120


@functools.lru_cache(maxsize=None)
def _make_gather(n_fields, f_base, embed, vocab, batch):
    info = plsc.get_sparse_core_info()
    nc, ns = info.num_cores, info.num_subcores
    nw = nc * ns
    egroups = embed // 8
    n_units = n_fields * egroups
    units_per_w = -(-n_units // nw)
    v_aligned = (vocab // 128) * 128
    n_full = v_aligned // _CV
    tail_v = v_aligned - n_full * _CV
    n_chunks = n_full + (1 if tail_v else 0)
    pos_bits = batch.bit_length() - 1
    assert (1 << pos_bits) == batch and vocab * batch < 2**31

    mesh = plsc.VectorSubcoreMesh(core_axis_name="c", subcore_axis_name="s")

    @functools.partial(
        pl.kernel,
        mesh=mesh,
        compiler_params=pltpu.CompilerParams(
            use_tc_tiling_on_sc=True, needs_layout_passes=False),
        out_type=jax.ShapeDtypeStruct((n_fields * embed, batch), jnp.float32),
        scratch_types=[
            pltpu.VMEM((batch,), jnp.int32),
            pltpu.VMEM((32,), jnp.int32),
            pltpu.VMEM((_CV // 128, 8, 128), jnp.float32),
            pltpu.VMEM((_CV // 128, 8, 128), jnp.float32),
            pltpu.VMEM((1, 8, 128), jnp.float32),
            pltpu.VMEM((8, batch), jnp.float32),
            pltpu.SemaphoreType.DMA,
            pltpu.SemaphoreType.DMA,
        ],
    )
    def gather_k(tbl_hbm, reg_hbm, sv_hbm, bnd_hbm, out_hbm,
                 sv_v, bnd_v, buf0, buf1, reg_v, slab_v, sem0, sem1):
        wid = lax.axis_index("s") * nc + lax.axis_index("c")
        bufs = (buf0, buf1)
        sems = (sem0, sem1)
        iota16 = lax.iota(jnp.int32, _LANE)

        def tile_copy(f, g8, c, t):
            v0 = pl.multiple_of(c * _CV + t * 128, 128)
            return pltpu.make_async_copy(
                tbl_hbm.at[f, pl.ds(g8, 8), pl.ds(v0, 128)],
                bufs[c % 2].at[t], sems[c % 2])

        def fire(f, g8, c):
            nt = (_CV if c < n_full else tail_v) // 128
            lax.fori_loop(0, nt, lambda t, a: (tile_copy(f, g8, c, t).start(), a)[1], 0)

        def wait(f, g8, c):
            nt = (_CV if c < n_full else tail_v) // 128
            lax.fori_loop(0, nt, lambda t, a: (tile_copy(f, g8, c, t).wait(), a)[1], 0)

        def scalar_at(vec_ref, j):
            grp = vec_ref[pl.ds((j // _LANE) * _LANE, _LANE)]
            return jnp.sum(jnp.where(iota16 == (j % _LANE), grp, 0))

        def unit_body(j, carry):
            u = wid + j * nw

            @pl.when(u < n_units)
            def _():
                fl = u // egroups
                f = fl + f_base
                g8 = pl.multiple_of((u % egroups) * 8, 8)
                pltpu.sync_copy(sv_hbm.at[fl], sv_v)
                pltpu.sync_copy(bnd_hbm.at[fl], bnd_v)
                pltpu.sync_copy(reg_hbm.at[fl, pl.ds(g8, 8), :], reg_v.at[0])

                def process(buf, v0, v1, lo, hi):
                    def group_body(k, c2):
                        off = pl.multiple_of(k * _LANE, 8)
                        pk16 = sv_v[pl.ds(off, _LANE)]
                        sv16 = lax.shift_right_logical(pk16, pos_bits)
                        pos16 = jnp.bitwise_and(pk16, batch - 1)
                        m = jnp.logical_and(sv16 >= v0, sv16 < v1)
                        vloc = jnp.where(m, sv16 - v0, 0)
                        t16 = lax.shift_right_logical(vloc, 7)
                        l16 = jnp.bitwise_and(vloc, 127)
                        for s in range(8):
                            s16 = jnp.full((_LANE,), s, jnp.int32)
                            vals = plsc.load_gather(buf, [t16, s16, l16])
                            plsc.store_scatter(slab_v, [s16, pos16], vals, mask=m)
                        return c2

                    lax.fori_loop(lo >> 4, (hi + _LANE - 1) >> 4, group_body, 0)

                fire(f, g8, 0)
                for c in range(n_chunks):
                    if c + 1 < n_chunks:
                        fire(f, g8, c + 1)
                    wait(f, g8, c)
                    v0 = c * _CV
                    v1 = v0 + (_CV if c < n_full else tail_v)
                    process(bufs[c % 2], v0, v1,
                            scalar_at(bnd_v, c), scalar_at(bnd_v, c + 1))
                process(reg_v, v_aligned, vocab,
                        scalar_at(bnd_v, n_chunks), scalar_at(bnd_v, n_chunks + 1))
                row0 = fl * embed + g8
                pltpu.sync_copy(slab_v, out_hbm.at[pl.ds(row0, 8), :])

            return carry

        lax.fori_loop(0, units_per_w, unit_body, 0)

    return gather_k


def _mlp_t_body(*refs):
    rt0_ref, rt1_ref = refs[0], refs[1]
    out_ref = refs[-1]
    w = refs[2:-1]
    rt = jnp.concatenate([rt0_ref[...], rt1_ref[...]], axis=0)
    cdims = (((0,), (0,)), ((), ()))

    def dot(a, b):
        return lax.dot_general(a, b, cdims, preferred_element_type=jnp.float32)

    n_units = (len(w) - 2) // 4
    for u in range(n_units):
        w1, b1, w2, b2 = w[4 * u : 4 * u + 4]
        h = jnp.maximum(dot(w1[...], rt) + b1[...], 0.0)
        h = dot(w2[...], h)
        rt = jnp.maximum(rt + h + b2[...], 0.0)
    wd, bd = w[-2], w[-1]
    logit = dot(wd[...], rt)
    out_ref[...] = jax.nn.sigmoid(logit + bd[...])


def _mlp_t(rt0, rt1, flat_w, block_c=512):
    batch = rt0.shape[1]
    grid = (batch // block_c,)
    full = lambda a: pl.BlockSpec(a.shape, lambda i: (0,) * a.ndim)
    in_specs = [pl.BlockSpec((rt0.shape[0], block_c), lambda i: (0, i)),
                pl.BlockSpec((rt1.shape[0], block_c), lambda i: (0, i))]
    in_specs += [full(a) for a in flat_w]
    return pl.pallas_call(
        _mlp_t_body,
        grid=grid,
        in_specs=in_specs,
        out_specs=pl.BlockSpec((1, block_c), lambda i: (0, i)),
        out_shape=jax.ShapeDtypeStruct((1, batch), jnp.float32),
    )(rt0, rt1, *flat_w)


def kernel(sparse_inputs, params):
    tables = params["tables"]
    n_fields, vocab, embed = tables.shape
    batch = sparse_inputs.shape[0]
    tbl_t = jnp.transpose(tables, (0, 2, 1))

    idx_t = sparse_inputs.astype(jnp.int32).T
    iota_b = jnp.broadcast_to(
        jnp.arange(batch, dtype=jnp.int32)[None, :], idx_t.shape)
    packed = idx_t * batch + iota_b
    v_aligned = (vocab // 128) * 128
    grid = jnp.array(
        list(range(0, v_aligned + 1, _CV))[: v_aligned // _CV + 1]
        + [v_aligned, vocab], dtype=jnp.int32)
    bnd = jnp.sum(idx_t[:, :, None] < grid[None, None, :], axis=1,
                  dtype=jnp.int32)
    bnd = jnp.pad(bnd, ((0, 0), (0, 32 - bnd.shape[1])), mode="edge")
    reg = jnp.pad(tbl_t[:, :, v_aligned:], ((0, 0), (0, 0),
                                            (0, 128 - (vocab - v_aligned))))

    fh = n_fields // 2
    flat_w = []
    for (w1, b1, w2, b2) in params["res"]:
        flat_w += [w1, b1[:, None], w2, b2[:, None]]
    flat_w += [params["Wd"], params["bd"][:, None]]

    halves = []
    for f0, f1 in ((0, fh), (fh, n_fields)):
        sv_h = lax.sort(packed[f0:f1], dimension=1)
        halves.append(_make_gather(f1 - f0, f0, embed, vocab, batch)(
            tbl_t, reg[f0:f1], sv_h, bnd[f0:f1]))
    out_t = _mlp_t(halves[0], halves[1], tuple(flat_w))
    return out_t.reshape(batch, 1)
